# Initial kernel scaffold; baseline (speedup 1.0000x reference)
import functools

import jax
import jax.numpy as jnp
from jax import lax
from jax.experimental import pallas as pl
from jax.experimental.pallas import tpu as pltpu
from jax.experimental.pallas import tpu_sc as plsc

_N_NODES = 10000
_N_EDGES = 160000
_D = 256
_DH = 128
_G = 64

_NC = 2
_NS = 16
_CH = 128
_NCHUNK = 80
_EPW = _CH * _NCHUNK
_E_PAD = _EPW * _NS
_TRASH = _N_NODES
_AGG_ROWS = _NS * 640

_EB = 1280
_NB = 1000


def _edge_t_body(ea_ref, we_ref, be_ref, out_ref):
    res = jnp.dot(ea_ref[...], we_ref[...], preferred_element_type=jnp.float32)
    res = res + be_ref[...]
    out_ref[0] = res[:, :_DH]
    out_ref[1] = res[:, _DH:]


def _edge_transform(edge_attr, We, be2):
    return pl.pallas_call(
        _edge_t_body,
        grid=(_N_EDGES // _EB,),
        in_specs=[
            pl.BlockSpec((_EB, 16), lambda i: (i, 0)),
            pl.BlockSpec((16, _D), lambda i: (0, 0)),
            pl.BlockSpec((1, _D), lambda i: (0, 0)),
        ],
        out_specs=pl.BlockSpec((2, _EB, _DH), lambda i: (0, i, 0)),
        out_shape=jax.ShapeDtypeStruct((2, _E_PAD, _DH), jnp.float32),
    )(edge_attr, We, be2)


_mesh = plsc.VectorSubcoreMesh(core_axis_name="c", subcore_axis_name="s")


@functools.partial(
    pl.kernel,
    out_type=jax.ShapeDtypeStruct((2, _N_NODES, _DH), jnp.float32),
    mesh=_mesh,
    scratch_types=[
        pltpu.VMEM((_CH,), jnp.int32),
        pltpu.VMEM((_CH,), jnp.int32),
        pltpu.VMEM((_CH, _DH), jnp.float32),
        pltpu.VMEM((_CH, _DH), jnp.float32),
        pltpu.VMEM_SHARED((_AGG_ROWS, _DH), jnp.float32),
        pltpu.SemaphoreType.DMA,
    ],
)
def _sc_edge_agg(x2, tcat, srcadj, dstr, out, srcv, dstv, xbuf, tbuf, aggsh, sem):
    ci = lax.axis_index("c")
    si = lax.axis_index("s")

    def zrow(e, carry):
        for jj in range(_DH // 16):
            xbuf[e, pl.ds(jj * 16, 16)] = jnp.zeros((16,), jnp.float32)
        return carry

    lax.fori_loop(0, _CH, zrow, 0)
    for k in range(5):
        pltpu.sync_copy(xbuf, aggsh.at[pl.ds(si * 640 + k * _CH, _CH)])
    plsc.subcore_barrier()

    def chunk(j, carry):
        base = si * _EPW + j * _CH
        pltpu.sync_copy(srcadj.at[ci, si, j], srcv)
        pltpu.sync_copy(dstr.at[si, j], dstv)
        pltpu.async_copy(x2.at[srcv], xbuf, sem).wait()
        pltpu.sync_copy(tcat.at[ci, pl.ds(base, _CH)], tbuf)

        def ebody(e, c2):
            for jj in range(_DH // 16):
                sl = pl.ds(jj * 16, 16)
                xbuf[e, sl] = jnp.maximum(xbuf[e, sl] + tbuf[e, sl], 0.0)
            return c2

        lax.fori_loop(0, _CH, ebody, 0)
        pltpu.sync_copy(xbuf, aggsh.at[dstv], add=True)
        return carry

    lax.fori_loop(0, _NCHUNK, chunk, 0)
    plsc.subcore_barrier()

    rows = _N_NODES // _NS
    pltpu.sync_copy(aggsh.at[pl.ds(si * rows, rows)],
                    out.at[ci, pl.ds(si * rows, rows)])


def _mlp_pool_body(x_ref, agg_ref, b_ref, w1_ref, b1_ref, w2_ref, b2_ref,
                   out_ref, sums, cnts):
    i = pl.program_id(0)

    @pl.when(i == 0)
    def _init():
        sums[...] = jnp.zeros_like(sums)
        cnts[...] = jnp.zeros_like(cnts)

    u = x_ref[...] + jnp.concatenate([agg_ref[0], agg_ref[1]], axis=1)
    h = jnp.maximum(
        jnp.dot(u, w1_ref[...], preferred_element_type=jnp.float32) + b1_ref[...],
        0.0)
    h = jnp.dot(h, w2_ref[...], preferred_element_type=jnp.float32) + b2_ref[...]

    bb = b_ref[0]
    gids = lax.broadcasted_iota(jnp.int32, (_G, _NB), 0)
    oh = (bb == gids).astype(jnp.float32)
    sums[...] += jnp.dot(oh, h, preferred_element_type=jnp.float32)
    cnts[...] += jnp.dot(oh, jnp.ones((_NB, _DH), jnp.float32),
                         preferred_element_type=jnp.float32)

    @pl.when(i == pl.num_programs(0) - 1)
    def _fin():
        c = jnp.maximum(cnts[...], 1.0)
        denom = jnp.concatenate([c, c], axis=1)
        out_ref[...] = jax.nn.sigmoid(sums[...] / denom)


def _mlp_pool(x, aggpair, batch_r, W1, b12, W2, b22):
    return pl.pallas_call(
        _mlp_pool_body,
        grid=(_N_NODES // _NB,),
        in_specs=[
            pl.BlockSpec((_NB, _D), lambda i: (i, 0)),
            pl.BlockSpec((2, _NB, _DH), lambda i: (0, i, 0)),
            pl.BlockSpec((1, 1, _NB), lambda i: (i, 0, 0)),
            pl.BlockSpec((_D, _D), lambda i: (0, 0)),
            pl.BlockSpec((1, _D), lambda i: (0, 0)),
            pl.BlockSpec((_D, _D), lambda i: (0, 0)),
            pl.BlockSpec((1, _D), lambda i: (0, 0)),
        ],
        out_specs=pl.BlockSpec((_G, _D), lambda i: (0, 0)),
        out_shape=jax.ShapeDtypeStruct((_G, _D), jnp.float32),
        scratch_shapes=[
            pltpu.VMEM((_G, _D), jnp.float32),
            pltpu.VMEM((_G, _DH), jnp.float32),
        ],
    )(x, aggpair, batch_r, W1, b12, W2, b22)


def kernel(x, edge_index, edge_attr, batch, We, be, W1, b1, W2, b2):
    src = edge_index[0]
    dst = edge_index[1]

    tcat = _edge_transform(edge_attr, We, be.reshape(1, _D))

    x2 = jnp.concatenate([x[:, :_DH], x[:, _DH:]], axis=0)

    pad = _E_PAD - _N_EDGES
    srcp = jnp.concatenate([src, jnp.zeros((pad,), jnp.int32)])
    dstp = jnp.concatenate([dst, jnp.full((pad,), _TRASH, jnp.int32)])
    srcadj = jnp.stack([srcp, srcp + _N_NODES]).reshape(2, _NS, _NCHUNK, _CH)
    dstr = dstp.reshape(_NS, _NCHUNK, _CH)

    aggpair = _sc_edge_agg(x2, tcat, srcadj, dstr)

    return _mlp_pool(x, aggpair, batch.reshape(_N_NODES // _NB, 1, _NB),
                     W1, b1.reshape(1, _D), W2, b2.reshape(1, _D))


# speedup vs baseline: 1.7467x; 1.7467x over previous
import functools

import jax
import jax.numpy as jnp
from jax import lax
from jax.experimental import pallas as pl
from jax.experimental.pallas import tpu as pltpu
from jax.experimental.pallas import tpu_sc as plsc

_N_NODES = 10000
_N_EDGES = 160000
_D = 256
_DH = 128
_G = 64

_NC = 2
_NS = 16
_CH = 128
_NCHUNK = 80
_EPW = _CH * _NCHUNK
_E_PAD = _EPW * _NS
_TRASH = _N_NODES
_AGG_ROWS = _NS * 640

_EB = 1280
_NB = 1000


def _edge_t_body(ea_ref, we_ref, be_ref, out_ref):
    res = jnp.dot(ea_ref[...], we_ref[...], preferred_element_type=jnp.float32)
    res = res + be_ref[...]
    out_ref[0] = res[:, :_DH]
    out_ref[1] = res[:, _DH:]


def _edge_transform(edge_attr, We, be2):
    return pl.pallas_call(
        _edge_t_body,
        grid=(_N_EDGES // _EB,),
        in_specs=[
            pl.BlockSpec((_EB, 16), lambda i: (i, 0)),
            pl.BlockSpec((16, _D), lambda i: (0, 0)),
            pl.BlockSpec((1, _D), lambda i: (0, 0)),
        ],
        out_specs=pl.BlockSpec((2, _EB, _DH), lambda i: (0, i, 0)),
        out_shape=jax.ShapeDtypeStruct((2, _E_PAD, _DH), jnp.float32),
    )(edge_attr, We, be2)


def _sc_edge_agg_body(x2, tcat, srcadj, dstr, out, srcv, dstv, xbuf, tbuf, aggsh, sem):
    ci = lax.axis_index("c")
    si = lax.axis_index("s")

    def zrow(e, carry):
        for jj in range(_DH // 16):
            xbuf[e, pl.ds(jj * 16, 16)] = jnp.zeros((16,), jnp.float32)
        return carry

    lax.fori_loop(0, _CH, zrow, 0)
    for k in range(5):
        pltpu.sync_copy(xbuf, aggsh.at[pl.ds(si * 640 + k * _CH, _CH)])
    plsc.subcore_barrier()

    def chunk(j, carry):
        base = si * _EPW + j * _CH
        pltpu.sync_copy(srcadj.at[ci, si, j], srcv)
        pltpu.sync_copy(dstr.at[si, j], dstv)
        pltpu.async_copy(x2.at[srcv], xbuf, sem).wait()
        pltpu.sync_copy(tcat.at[ci, pl.ds(base, _CH)], tbuf)

        def ebody(e, c2):
            for jj in range(_DH // 16):
                sl = pl.ds(jj * 16, 16)
                xbuf[e, sl] = jnp.maximum(xbuf[e, sl] + tbuf[e, sl], 0.0)
            return c2

        lax.fori_loop(0, _CH, ebody, 0)
        pltpu.sync_copy(xbuf, aggsh.at[dstv], add=True)
        return carry

    lax.fori_loop(0, _NCHUNK, chunk, 0)
    plsc.subcore_barrier()

    pltpu.sync_copy(aggsh.at[pl.ds(si * 640, 640)],
                    out.at[ci, pl.ds(si * 640, 640)])


@functools.cache
def _sc_edge_agg():
    mesh = plsc.VectorSubcoreMesh(core_axis_name="c", subcore_axis_name="s",
                                  num_cores=_NC, num_subcores=_NS)
    return pl.kernel(
        _sc_edge_agg_body,
        out_type=jax.ShapeDtypeStruct((2, _AGG_ROWS, _DH), jnp.float32),
        mesh=mesh,
        scratch_types=[
            pltpu.VMEM((_CH,), jnp.int32),
            pltpu.VMEM((_CH,), jnp.int32),
            pltpu.VMEM((_CH, _DH), jnp.float32),
            pltpu.VMEM((_CH, _DH), jnp.float32),
            pltpu.VMEM_SHARED((_AGG_ROWS, _DH), jnp.float32),
            pltpu.SemaphoreType.DMA,
        ],
    )


def _mlp_pool_body(x_ref, agg_ref, b_ref, w1_ref, b1_ref, w2_ref, b2_ref,
                   out_ref, sums, cnts):
    i = pl.program_id(0)

    @pl.when(i == 0)
    def _init():
        sums[...] = jnp.zeros_like(sums)
        cnts[...] = jnp.zeros_like(cnts)

    u = x_ref[...] + jnp.concatenate([agg_ref[0], agg_ref[1]], axis=1)
    h = jnp.maximum(
        jnp.dot(u, w1_ref[...], preferred_element_type=jnp.float32) + b1_ref[...],
        0.0)
    h = jnp.dot(h, w2_ref[...], preferred_element_type=jnp.float32) + b2_ref[...]

    bb = b_ref[0]
    gids = lax.broadcasted_iota(jnp.int32, (_G, _NB), 0)
    oh = (bb == gids).astype(jnp.float32)
    sums[...] += jnp.dot(oh, h, preferred_element_type=jnp.float32)
    cnts[...] += jnp.dot(oh, jnp.ones((_NB, _DH), jnp.float32),
                         preferred_element_type=jnp.float32)

    @pl.when(i == pl.num_programs(0) - 1)
    def _fin():
        c = jnp.maximum(cnts[...], 1.0)
        denom = jnp.concatenate([c, c], axis=1)
        out_ref[...] = jax.nn.sigmoid(sums[...] / denom)


def _mlp_pool(x, aggpair, batch_r, W1, b12, W2, b22):
    return pl.pallas_call(
        _mlp_pool_body,
        grid=(_N_NODES // _NB,),
        in_specs=[
            pl.BlockSpec((_NB, _D), lambda i: (i, 0)),
            pl.BlockSpec((2, _NB, _DH), lambda i: (0, i, 0)),
            pl.BlockSpec((1, 1, _NB), lambda i: (i, 0, 0)),
            pl.BlockSpec((_D, _D), lambda i: (0, 0)),
            pl.BlockSpec((1, _D), lambda i: (0, 0)),
            pl.BlockSpec((_D, _D), lambda i: (0, 0)),
            pl.BlockSpec((1, _D), lambda i: (0, 0)),
        ],
        out_specs=pl.BlockSpec((_G, _D), lambda i: (0, 0)),
        out_shape=jax.ShapeDtypeStruct((_G, _D), jnp.float32),
        scratch_shapes=[
            pltpu.VMEM((_G, _D), jnp.float32),
            pltpu.VMEM((_G, _DH), jnp.float32),
        ],
    )(x, aggpair, batch_r, W1, b12, W2, b22)


def kernel(x, edge_index, edge_attr, batch, We, be, W1, b1, W2, b2):
    src = edge_index[0]
    dst = edge_index[1]

    tcat = _edge_transform(edge_attr, We, be.reshape(1, _D))

    x2 = jnp.concatenate([x[:, :_DH], x[:, _DH:]], axis=0)

    pad = _E_PAD - _N_EDGES
    srcp = jnp.concatenate([src, jnp.zeros((pad,), jnp.int32)])
    dstp = jnp.concatenate([dst, jnp.full((pad,), _TRASH, jnp.int32)])
    srcadj = jnp.stack([srcp, srcp + _N_NODES]).reshape(2, _NS, _NCHUNK, _CH)
    dstr = dstp.reshape(_NS, _NCHUNK, _CH)

    aggpair = _sc_edge_agg()(x2, tcat, srcadj, dstr)

    return _mlp_pool(x, aggpair, batch.reshape(_N_NODES // _NB, 1, _NB),
                     W1, b1.reshape(1, _D), W2, b2.reshape(1, _D))


# speedup vs baseline: 3.3711x; 1.9300x over previous
import functools

import jax
import jax.numpy as jnp
from jax import lax
from jax.experimental import pallas as pl
from jax.experimental.pallas import tpu as pltpu
from jax.experimental.pallas import tpu_sc as plsc

_N_NODES = 10000
_N_EDGES = 160000
_D = 256
_DH = 128
_G = 64

_NC = 2
_NS = 16
_CH = 80
_NCHUNK = 125
_EPW = _CH * _NCHUNK
_AGG_ROWS = _NS * 640

_EB = 1280
_NB = 1000


def _edge_t_body(ea_ref, we_ref, be_ref, out_ref):
    res = jnp.dot(ea_ref[...], we_ref[...], preferred_element_type=jnp.float32)
    res = res + be_ref[...]
    out_ref[0] = res[:, :_DH]
    out_ref[1] = res[:, _DH:]


def _edge_transform(edge_attr, We, be2):
    return pl.pallas_call(
        _edge_t_body,
        grid=(_N_EDGES // _EB,),
        in_specs=[
            pl.BlockSpec((_EB, 16), lambda i: (i, 0)),
            pl.BlockSpec((16, _D), lambda i: (0, 0)),
            pl.BlockSpec((1, _D), lambda i: (0, 0)),
        ],
        out_specs=pl.BlockSpec((2, _EB, _DH), lambda i: (0, i, 0)),
        out_shape=jax.ShapeDtypeStruct((2, _N_EDGES, _DH), jnp.float32),
    )(edge_attr, We, be2)


def _sc_edge_agg_body(x2, tcat, idxc, out,
                      idxbuf, xbuf, tbuf, aggsh,
                      gsem0, gsem1, tsem0, tsem1,
                      isem0, isem1, isem2, isem3):
    ci = lax.axis_index("c")
    si = lax.axis_index("s")
    gsems = (gsem0, gsem1)
    tsems = (tsem0, tsem1)
    isems = (isem0, isem1, isem2, isem3)

    def t_src(j):
        base = pl.multiple_of(si * _EPW + j * _CH, _CH)
        return tcat.at[ci, pl.ds(base, _CH)]

    def zrow(e, carry):
        for k in range(_DH // 16):
            tbuf[0, e, pl.ds(k * 16, 16)] = jnp.zeros((16,), jnp.float32)
        return carry

    lax.fori_loop(0, _CH, zrow, 0)
    for k in range(8):
        pltpu.sync_copy(tbuf.at[0], aggsh.at[pl.ds(si * 640 + k * _CH, _CH)])

    for s in range(4):
        pltpu.async_copy(idxc.at[ci, si, s], idxbuf.at[s], isems[s])
    for b in range(2):
        pltpu.make_async_copy(idxc.at[ci, si, b], idxbuf.at[b],
                              isems[b]).wait()
        pltpu.async_copy(x2.at[idxbuf.at[b, 0]], xbuf.at[b], gsems[b])
        pltpu.async_copy(t_src(b), tbuf.at[b], tsems[b])

    plsc.subcore_barrier()

    def _compute(db):
        def ebody(e, c2):
            for k in range(_DH // 16):
                sl = pl.ds(k * 16, 16)
                xbuf[db, e, sl] = jnp.maximum(
                    xbuf[db, e, sl] + tbuf[db, e, sl], 0.0)
            return c2

        lax.fori_loop(0, _CH, ebody, 0)

    def _step(j, b, tail):
        db = b % 2
        s_cur = b
        s_next = (b + 2) % 4
        pltpu.make_async_copy(x2.at[idxbuf.at[s_cur, 0]], xbuf.at[db],
                              gsems[db]).wait()
        pltpu.make_async_copy(t_src(j), tbuf.at[db], tsems[db]).wait()
        _compute(db)
        pltpu.sync_copy(xbuf.at[db], aggsh.at[idxbuf.at[s_cur, 1]], add=True)
        if tail:
            return

        nj = j + 2

        @pl.when(nj < _NCHUNK)
        def _issue_data():
            pltpu.make_async_copy(idxc.at[ci, si, nj], idxbuf.at[s_next],
                                  isems[s_next]).wait()
            pltpu.async_copy(x2.at[idxbuf.at[s_next, 0]], xbuf.at[db],
                             gsems[db])
            pltpu.async_copy(t_src(nj), tbuf.at[db], tsems[db])

        fj = j + 4

        @pl.when(fj < _NCHUNK)
        def _refill_idx():
            pltpu.async_copy(idxc.at[ci, si, fj], idxbuf.at[s_cur],
                             isems[s_cur])

    def quad(jj, carry):
        j0 = jj * 4
        for b in range(4):
            _step(j0 + b, b, tail=False)
        return carry

    lax.fori_loop(0, (_NCHUNK - 1) // 4, quad, 0)
    _step(_NCHUNK - 1, 0, tail=True)

    plsc.subcore_barrier()

    pltpu.sync_copy(aggsh.at[pl.ds(si * 640, 640)],
                    out.at[ci, pl.ds(si * 640, 640)])


@functools.cache
def _sc_edge_agg():
    mesh = plsc.VectorSubcoreMesh(core_axis_name="c", subcore_axis_name="s",
                                  num_cores=_NC, num_subcores=_NS)
    return pl.kernel(
        _sc_edge_agg_body,
        out_type=jax.ShapeDtypeStruct((2, _AGG_ROWS, _DH), jnp.float32),
        mesh=mesh,
        scratch_types=[
            pltpu.VMEM((4, 2, _CH), jnp.int32),
            pltpu.VMEM((2, _CH, _DH), jnp.float32),
            pltpu.VMEM((2, _CH, _DH), jnp.float32),
            pltpu.VMEM_SHARED((_AGG_ROWS, _DH), jnp.float32),
            pltpu.SemaphoreType.DMA,
            pltpu.SemaphoreType.DMA,
            pltpu.SemaphoreType.DMA,
            pltpu.SemaphoreType.DMA,
            pltpu.SemaphoreType.DMA,
            pltpu.SemaphoreType.DMA,
            pltpu.SemaphoreType.DMA,
            pltpu.SemaphoreType.DMA,
        ],
    )


def _mlp_pool_body(x_ref, agg_ref, b_ref, w1_ref, b1_ref, w2_ref, b2_ref,
                   out_ref, sums, cnts):
    i = pl.program_id(0)

    @pl.when(i == 0)
    def _init():
        sums[...] = jnp.zeros_like(sums)
        cnts[...] = jnp.zeros_like(cnts)

    u = x_ref[...] + jnp.concatenate([agg_ref[0], agg_ref[1]], axis=1)
    h = jnp.maximum(
        jnp.dot(u, w1_ref[...], preferred_element_type=jnp.float32) + b1_ref[...],
        0.0)
    h = jnp.dot(h, w2_ref[...], preferred_element_type=jnp.float32) + b2_ref[...]

    bb = b_ref[0]
    gids = lax.broadcasted_iota(jnp.int32, (_G, _NB), 0)
    oh = (bb == gids).astype(jnp.float32)
    sums[...] += jnp.dot(oh, h, preferred_element_type=jnp.float32)
    cnts[...] += jnp.dot(oh, jnp.ones((_NB, _DH), jnp.float32),
                         preferred_element_type=jnp.float32)

    @pl.when(i == pl.num_programs(0) - 1)
    def _fin():
        c = jnp.maximum(cnts[...], 1.0)
        denom = jnp.concatenate([c, c], axis=1)
        out_ref[...] = jax.nn.sigmoid(sums[...] / denom)


def _mlp_pool(x, aggpair, batch_r, W1, b12, W2, b22):
    return pl.pallas_call(
        _mlp_pool_body,
        grid=(_N_NODES // _NB,),
        in_specs=[
            pl.BlockSpec((_NB, _D), lambda i: (i, 0)),
            pl.BlockSpec((2, _NB, _DH), lambda i: (0, i, 0)),
            pl.BlockSpec((1, 1, _NB), lambda i: (i, 0, 0)),
            pl.BlockSpec((_D, _D), lambda i: (0, 0)),
            pl.BlockSpec((1, _D), lambda i: (0, 0)),
            pl.BlockSpec((_D, _D), lambda i: (0, 0)),
            pl.BlockSpec((1, _D), lambda i: (0, 0)),
        ],
        out_specs=pl.BlockSpec((_G, _D), lambda i: (0, 0)),
        out_shape=jax.ShapeDtypeStruct((_G, _D), jnp.float32),
        scratch_shapes=[
            pltpu.VMEM((_G, _D), jnp.float32),
            pltpu.VMEM((_G, _DH), jnp.float32),
        ],
    )(x, aggpair, batch_r, W1, b12, W2, b22)


def kernel(x, edge_index, edge_attr, batch, We, be, W1, b1, W2, b2):
    src = edge_index[0]
    dst = edge_index[1]

    tcat = _edge_transform(edge_attr, We, be.reshape(1, _D))

    x2 = jnp.concatenate([x[:, :_DH], x[:, _DH:]], axis=0)

    src_r = src.reshape(_NS, _NCHUNK, _CH)
    dst_r = dst.reshape(_NS, _NCHUNK, _CH)
    idxc = jnp.stack([
        jnp.stack([src_r, dst_r], axis=2),
        jnp.stack([src_r + _N_NODES, dst_r], axis=2),
    ])

    aggpair = _sc_edge_agg()(x2, tcat, idxc)

    return _mlp_pool(x, aggpair, batch.reshape(_N_NODES // _NB, 1, _NB),
                     W1, b1.reshape(1, _D), W2, b2.reshape(1, _D))
